# baseline (speedup 1.0000x reference)
import functools

import jax
import jax.numpy as jnp
import numpy as np
from jax import lax
from jax.experimental import pallas as pl
from jax.experimental.pallas import tpu as pltpu
from jax.experimental.pallas import tpu_sc as plsc

_DIM = 64
_PAD = 128
_BATCH = 16384
_FIELDS = 26
_FPAD = 32

_NC = 2
_NS = 16
_NW = _NC * _NS
_TOTAL = _BATCH * _FPAD
_PER_W = _TOTAL // _NW
_C = 128
_NCH = _PER_W // _C
_GSUB = 2
_G = _C * _GSUB
_NBIG = _PER_W // _G
_DEPTH = 3


def _make_gather():
    mesh = plsc.VectorSubcoreMesh(core_axis_name="c", subcore_axis_name="s")

    @functools.partial(
        pl.kernel,
        mesh=mesh,
        compiler_params=pltpu.CompilerParams(use_tc_tiling_on_sc=True),
        out_type=jax.ShapeDtypeStruct((_TOTAL, _PAD), jnp.float32),
        scratch_types=[
            pltpu.VMEM((_NCH, _C), jnp.int32),
            pltpu.VMEM((_DEPTH, _G, _PAD), jnp.float32),
            pltpu.SemaphoreType.DMA((_DEPTH,)),
        ],
    )
    def gather_k(table_hbm, idx_hbm, out_hbm, idx_v, rows_v, sems):
        wid = lax.axis_index("s") * _NC + lax.axis_index("c")
        pltpu.sync_copy(idx_hbm.at[wid], idx_v)
        base = wid * _PER_W

        def fire(g, b):
            for j in range(_GSUB):
                pltpu.async_copy(
                    table_hbm.at[idx_v.at[g * _GSUB + j]],
                    rows_v.at[b, pl.ds(j * _C, _C)],
                    sems.at[b],
                )

        for d in range(_DEPTH):
            fire(d, d)

        def body(g, carry):
            b = lax.rem(g, _DEPTH)
            pltpu.make_async_copy(
                out_hbm.at[pl.ds(0, _G)], rows_v.at[b], sems.at[b]
            ).wait()
            pltpu.sync_copy(rows_v.at[b], out_hbm.at[pl.ds(base + g * _G, _G)])

            @pl.when(g + _DEPTH < _NBIG)
            def _():
                fire(g + _DEPTH, b)

            return carry

        lax.fori_loop(0, _NBIG, body, 0)

    return gather_k


_gather = _make_gather()

_WIDEN = np.eye(_DIM, _PAD, dtype=np.float32)


@jax.jit
def kernel(batch, embeddings):
    b32 = batch.astype(jnp.int32)
    idx = jnp.concatenate([b32, b32[:, : _FPAD - _FIELDS]], axis=1)
    idx = idx.reshape(_NW, _NCH, _C)
    table = lax.dot_general(
        embeddings,
        _WIDEN,
        (((1,), (0,)), ((), ())),
        precision=lax.Precision.HIGH,
    )
    out = _gather(table, idx)
    out = out.reshape(_BATCH, _FPAD, _PAD)
    return out[:, :_FIELDS, :_DIM]


# speedup vs baseline: 1.0213x; 1.0213x over previous
import functools

import jax
import jax.numpy as jnp
import numpy as np
from jax import lax
from jax.experimental import pallas as pl
from jax.experimental.pallas import tpu as pltpu
from jax.experimental.pallas import tpu_sc as plsc

_DIM = 64
_PAD = 128
_BATCH = 16384
_FIELDS = 26
_FPAD = 32

_NC = 2
_NS = 16
_NW = _NC * _NS
_TOTAL = _BATCH * _FPAD
_PER_W = _TOTAL // _NW
_C = 128
_NCH = _PER_W // _C
_GSUB = 2
_G = _C * _GSUB
_NBIG = _PER_W // _G
_DEPTH = 3


def _make_gather():
    mesh = plsc.VectorSubcoreMesh(core_axis_name="c", subcore_axis_name="s")

    @functools.partial(
        pl.kernel,
        mesh=mesh,
        compiler_params=pltpu.CompilerParams(use_tc_tiling_on_sc=True),
        out_type=jax.ShapeDtypeStruct((_TOTAL, _PAD), jnp.float32),
        scratch_types=[
            pltpu.VMEM((_NCH, _C), jnp.int32),
            pltpu.VMEM((_DEPTH, _G, _PAD), jnp.float32),
            pltpu.SemaphoreType.DMA((_DEPTH,)),
        ],
    )
    def gather_k(table_hbm, idx_hbm, out_hbm, idx_v, rows_v, sems):
        wid = lax.axis_index("s") * _NC + lax.axis_index("c")
        pltpu.sync_copy(idx_hbm.at[wid], idx_v)
        base = wid * _PER_W

        def fire(g, b):
            for j in range(_GSUB * _C // _FPAD):
                pltpu.async_copy(
                    table_hbm.at[
                        idx_v.at[g * _GSUB + j // 4, pl.ds(j % 4 * _FPAD, _FIELDS)]
                    ],
                    rows_v.at[b, pl.ds(j * _FPAD, _FIELDS)],
                    sems.at[b],
                )

        for d in range(_DEPTH):
            fire(d, d)

        def body(g, carry):
            b = lax.rem(g, _DEPTH)
            nreal = _G // _FPAD * _FIELDS
            pltpu.make_async_copy(
                out_hbm.at[pl.ds(0, nreal)],
                rows_v.at[b, pl.ds(0, nreal)],
                sems.at[b],
            ).wait()
            pltpu.sync_copy(rows_v.at[b], out_hbm.at[pl.ds(base + g * _G, _G)])

            @pl.when(g + _DEPTH < _NBIG)
            def _():
                fire(g + _DEPTH, b)

            return carry

        lax.fori_loop(0, _NBIG, body, 0)

    return gather_k


_gather = _make_gather()

_WIDEN = np.eye(_DIM, _PAD, dtype=np.float32)


@jax.jit
def kernel(batch, embeddings):
    b32 = batch.astype(jnp.int32)
    idx = jnp.concatenate([b32, b32[:, : _FPAD - _FIELDS]], axis=1)
    idx = idx.reshape(_NW, _NCH, _C)
    table = lax.dot_general(
        embeddings,
        _WIDEN,
        (((1,), (0,)), ((), ())),
        precision=lax.Precision.HIGH,
    )
    out = _gather(table, idx)
    out = out.reshape(_BATCH, _FPAD, _PAD)
    return out[:, :_FIELDS, :_DIM]
